# Initial kernel scaffold; baseline (speedup 1.0000x reference)
import jax
import jax.numpy as jnp
from jax.experimental import pallas as pl


def kernel(input_mixed, ref_panel):
    raise NotImplementedError("write your pallas kernel here")


# speedup vs baseline: 35.5565x; 35.5565x over previous
import functools

import jax
import jax.numpy as jnp
from jax import lax
from jax.experimental import pallas as pl
from jax.experimental.pallas import tpu as pltpu
from jax.experimental.pallas import tpu_sc as plsc

B, C, N, L = 4, 3, 512, 4096
BC = B * C
NW = 32
COLS = L // NW
LANES = 16
GRP = COLS // LANES
RBLK = 128
NBLK = N // RBLK
NTASK = BC * NBLK
NEG = float("-inf")


def _sc_topk(mixed, ref3):
    mesh = plsc.VectorSubcoreMesh(core_axis_name="c", subcore_axis_name="s")

    @functools.partial(
        pl.kernel,
        mesh=mesh,
        out_type=[
            jax.ShapeDtypeStruct((BC, 2, L), jnp.float32),
            jax.ShapeDtypeStruct((BC, L), jnp.int32),
        ],
        scratch_types=[
            pltpu.VMEM((B, L), jnp.float32),
            pltpu.VMEM((2, RBLK, COLS), jnp.float32),
            pltpu.VMEM((2, COLS), jnp.float32),
            pltpu.VMEM((COLS,), jnp.int32),
            pltpu.SemaphoreType.DMA,
            pltpu.SemaphoreType.DMA,
        ],
    )
    def k(mixed_hbm, ref_hbm, maxs_hbm, idxs_hbm, m_v, buf_v, ov_v, oi_v,
          sem0, sem1):
        wid = lax.axis_index("s") * 2 + lax.axis_index("c")
        col0 = wid * COLS
        sems = (sem0, sem1)

        def task_copy(t, slot):
            bc = t // NBLK
            blk = t % NBLK
            return pltpu.make_async_copy(
                ref_hbm.at[bc, pl.ds(blk * RBLK, RBLK), pl.ds(col0, COLS)],
                buf_v.at[slot],
                sems[slot],
            )

        pltpu.sync_copy(mixed_hbm, m_v)
        task_copy(0, 0).start()
        task_copy(1, 1).start()

        def outer(tp, carry):
            for u in range(2):
                t = tp * 2 + u
                bc = t // NBLK
                blk = t % NBLK
                b = bc // C
                task_copy(t, u).wait()

                def _fresh():
                    out = []
                    for _g in range(GRP):
                        out.append(jnp.full((LANES,), NEG, jnp.float32))
                        out.append(jnp.full((LANES,), NEG, jnp.float32))
                        out.append(jnp.zeros((LANES,), jnp.int32))
                    return tuple(out)

                st = lax.cond(blk == 0, _fresh, lambda: tuple(carry))
                mv = [m_v[b, pl.ds(col0 + g * LANES, LANES)]
                      for g in range(GRP)]

                def inner(i, s):
                    nvec = jnp.full((LANES,), blk * RBLK + i, jnp.int32)
                    out = []
                    for g in range(GRP):
                        m1 = s[3 * g]
                        m2 = s[3 * g + 1]
                        ix = s[3 * g + 2]
                        v = buf_v[u, i, pl.ds(g * LANES, LANES)] * mv[g]
                        gt = v > m1
                        m2 = jnp.where(gt, m1, jnp.maximum(m2, v))
                        ix = jnp.where(gt, nvec, ix)
                        m1 = jnp.where(gt, v, m1)
                        out += [m1, m2, ix]
                    return tuple(out)

                carry = lax.fori_loop(0, RBLK, inner, st)

                nt = t + 2

                @pl.when(nt < NTASK)
                def _start_next(t=nt, u=u):
                    task_copy(t, u).start()

                @pl.when(blk == NBLK - 1)
                def _flush(carry=carry, bc=bc):
                    for g in range(GRP):
                        ov_v[0, pl.ds(g * LANES, LANES)] = carry[3 * g]
                        ov_v[1, pl.ds(g * LANES, LANES)] = carry[3 * g + 1]
                        oi_v[pl.ds(g * LANES, LANES)] = carry[3 * g + 2]
                    pltpu.sync_copy(ov_v, maxs_hbm.at[bc, :, pl.ds(col0, COLS)])
                    pltpu.sync_copy(oi_v, idxs_hbm.at[bc, pl.ds(col0, COLS)])

            return carry

        init = []
        for _g in range(GRP):
            init.append(jnp.full((LANES,), NEG, jnp.float32))
            init.append(jnp.full((LANES,), NEG, jnp.float32))
            init.append(jnp.zeros((LANES,), jnp.int32))
        lax.fori_loop(0, NTASK // 2, outer, tuple(init))

    return k(mixed, ref3)


def kernel(input_mixed, ref_panel):
    ref3 = ref_panel.reshape(BC, N, L)
    maxs, idxs = _sc_topk(input_mixed, ref3)
    return maxs.reshape(B, C, 2, L), idxs.reshape(B, C, L)
